# baseline (speedup 1.0000x reference)
import functools

import jax
import jax.numpy as jnp
from jax import lax
from jax.experimental import pallas as pl
from jax.experimental.pallas import tpu as pltpu
from jax.experimental.pallas import tpu_sc as plsc

N0, N1, N2 = 50000, 10000, 2000
D = 256
HALF = 128
E1, E2 = 160000, 32000

NC, NS = 2, 16
CHUNK = 128
NBUF = 4
NACC = 2048
PBITS = 14
PMASK = (1 << PBITS) - 1
SCRAP = NACC - 1


def _make_sc_agg(eps, table_rows, resident):
    astripe = NACC // NS
    tstripe = table_rows // NS
    mesh = plsc.VectorSubcoreMesh(
        core_axis_name="c", subcore_axis_name="s", num_cores=NC,
        num_subcores=NS)

    scratch = [
        pltpu.VMEM((eps,), jnp.int32),
        pltpu.VMEM((eps + NBUF * CHUNK + 16,), jnp.int32),
        [pltpu.VMEM((CHUNK,), jnp.int32) for _ in range(NBUF)],
        [pltpu.VMEM((CHUNK,), jnp.int32) for _ in range(NBUF)],
        [pltpu.VMEM((CHUNK, HALF), jnp.float32) for _ in range(NBUF)],
        pltpu.VMEM_SHARED((NACC, HALF), jnp.float32),
        [pltpu.SemaphoreType.DMA for _ in range(NBUF)],
    ]
    if resident:
        scratch.append(pltpu.VMEM_SHARED((table_rows, HALF), jnp.float32))

    @functools.partial(
        pl.kernel,
        out_type=jax.ShapeDtypeStruct((NC * NACC, HALF), jnp.float32),
        mesh=mesh,
        scratch_types=scratch,
        compiler_params=pltpu.CompilerParams(needs_layout_passes=False),
    )
    def sc_agg(table_hbm, pidx_hbm, zeros_hbm, out_hbm,
               pidx_v, comp_v, gidx_bufs, dst_bufs, rows_bufs,
               acc_sh, gsems, *maybe_tab):
        c = lax.axis_index("c")
        s = lax.axis_index("s")
        table = maybe_tab[0] if resident else table_hbm
        pltpu.sync_copy(pidx_hbm.at[s], pidx_v)
        pltpu.sync_copy(zeros_hbm, acc_sh.at[pl.ds(s * astripe, astripe)])
        if resident:
            pltpu.sync_copy(
                table_hbm.at[pl.ds(s * tstripe, tstripe)],
                maybe_tab[0].at[pl.ds(s * tstripe, tstripe)])
        plsc.subcore_barrier()

        trash = eps + NBUF * CHUNK + 15

        def scan_body(i, ptr):
            p16 = pidx_v[pl.ds(i * 16, 16)]
            keep = (p16 & PMASK) < N2
            prefix = plsc.cumsum(keep.astype(jnp.int32))
            pos = jnp.where(keep, ptr + prefix - 1, trash)
            plsc.store_scatter(comp_v, [pos], p16)
            return ptr + prefix[15]

        cnt = lax.fori_loop(0, eps // 16, scan_body, jnp.int32(0))
        for k in range(NBUF * CHUNK // 16):
            comp_v[pl.ds(cnt + k * 16, 16)] = jnp.full((16,), SCRAP, jnp.int32)
        n_ch = (cnt + CHUNK - 1) // CHUNK
        n_cp = jnp.maximum((n_ch + NBUF - 1) // NBUF * NBUF, NBUF)

        def prep_idx(j, b):
            for k in range(CHUNK // 16):
                p16 = comp_v[pl.ds(j * CHUNK + k * 16, 16)]
                gidx_bufs[b][pl.ds(k * 16, 16)] = 2 * (p16 >> PBITS) + c
                dst_bufs[b][pl.ds(k * 16, 16)] = p16 & PMASK

        def start_gather(b, sem):
            pltpu.async_copy(table.at[gidx_bufs[b]], rows_bufs[b], sem)

        def stage(j, b, sem):
            pltpu.make_async_copy(
                table.at[gidx_bufs[b]], rows_bufs[b], sem).wait()
            pltpu.sync_copy(rows_bufs[b], acc_sh.at[dst_bufs[b]], add=True)

            @pl.when(j + NBUF < n_cp)
            def _():
                prep_idx(j + NBUF, b)
                start_gather(b, sem)

        for b in range(NBUF):
            prep_idx(jnp.int32(b), b)
            start_gather(b, gsems[b])

        def ring_body(p, carry):
            for b in range(NBUF):
                stage(NBUF * p + b, b, gsems[b])
            return carry

        lax.fori_loop(0, n_cp // NBUF, ring_body, 0)
        plsc.subcore_barrier()
        pltpu.sync_copy(acc_sh.at[pl.ds(s * astripe, astripe)],
                        out_hbm.at[pl.ds(c * NACC + s * astripe, astripe)])

    return sc_agg


def _prep_edges(edge_index, eps):
    e = edge_index.astype(jnp.int32)
    pad = NS * eps - e.shape[1]
    p = (e[0] << PBITS) | e[1]
    p = jnp.concatenate([p, jnp.full((pad,), PMASK, jnp.int32)])
    return p.reshape(NS, eps)


def _mlp_body(x_ref, a0_ref, a1_ref, w1_ref, b1_ref, w2_ref, b2_ref, o_ref):
    h = x_ref[...] + jnp.concatenate([a0_ref[...], a1_ref[...]], axis=1)
    a = jnp.maximum(
        jnp.dot(h, w1_ref[...], preferred_element_type=jnp.float32)
        + b1_ref[...], 0.0)
    o_ref[...] = jnp.maximum(
        jnp.dot(a, w2_ref[...], preferred_element_type=jnp.float32)
        + b2_ref[...], 0.0)


def _mlp(x, aggbuf, n_rows, blk, W1, b1, W2, b2):
    grid = (NACC // blk,)
    nblk_off = NACC // blk
    return pl.pallas_call(
        _mlp_body,
        grid=grid,
        in_specs=[
            pl.BlockSpec((blk, D), lambda i: (i, 0)),
            pl.BlockSpec((blk, HALF), lambda i: (i, 0)),
            pl.BlockSpec((blk, HALF), lambda i, o=nblk_off: (o + i, 0)),
            pl.BlockSpec((D, D), lambda i: (0, 0)),
            pl.BlockSpec((D,), lambda i: (0,)),
            pl.BlockSpec((D, D), lambda i: (0, 0)),
            pl.BlockSpec((D,), lambda i: (0,)),
        ],
        out_specs=pl.BlockSpec((blk, D), lambda i: (i, 0)),
        out_shape=jax.ShapeDtypeStruct((n_rows, D), jnp.float32),
    )(x, aggbuf, aggbuf, W1, b1, W2, b2)


_EPS1 = 10240
_EPS2 = 2048
_H1ROWS = 2048

_sc_agg1 = _make_sc_agg(_EPS1, 2 * N1, resident=False)
_sc_agg2 = _make_sc_agg(_EPS2, 2 * _H1ROWS, resident=True)


def kernel(x, edge_index1, edge_index2, W1a, b1a, W2a, b2a,
           W1b, b1b, W2b, b2b):
    pidx1 = _prep_edges(edge_index1, _EPS1)
    pidx2 = _prep_edges(edge_index2, _EPS2)
    zeros = jnp.zeros((NACC // NS, HALF), jnp.float32)

    x2 = x.reshape(2 * N0, HALF)
    agg1 = _sc_agg1(x2, pidx1, zeros)
    h1 = _mlp(x, agg1, _H1ROWS, 512, W1a, b1a, W2a, b2a)

    h1_2 = h1.reshape(2 * _H1ROWS, HALF)
    agg2 = _sc_agg2(h1_2, pidx2, zeros)
    h2 = _mlp(h1, agg2, N2, 512, W1b, b1b, W2b, b2b)
    return h2


# speedup vs baseline: 1.0064x; 1.0064x over previous
import functools

import jax
import jax.numpy as jnp
from jax import lax
from jax.experimental import pallas as pl
from jax.experimental.pallas import tpu as pltpu
from jax.experimental.pallas import tpu_sc as plsc

N0, N1, N2 = 50000, 10000, 2000
D = 256
HALF = 128
E1, E2 = 160000, 32000

NC, NS = 2, 16
NACC = 2048
PBITS = 14
PMASK = (1 << PBITS) - 1
SCRAP = NACC - 1


def _make_sc_agg(eps, table_rows, resident, CHUNK, NBUF):
    astripe = NACC // NS
    tstripe = table_rows // NS
    mesh = plsc.VectorSubcoreMesh(
        core_axis_name="c", subcore_axis_name="s", num_cores=NC,
        num_subcores=NS)

    scratch = [
        pltpu.VMEM((eps,), jnp.int32),
        pltpu.VMEM((eps + NBUF * CHUNK + 16,), jnp.int32),
        [pltpu.VMEM((CHUNK,), jnp.int32) for _ in range(NBUF)],
        [pltpu.VMEM((CHUNK,), jnp.int32) for _ in range(NBUF)],
        [pltpu.VMEM((CHUNK, HALF), jnp.float32) for _ in range(NBUF)],
        pltpu.VMEM_SHARED((NACC, HALF), jnp.float32),
        [pltpu.SemaphoreType.DMA for _ in range(NBUF)],
    ]
    if resident:
        scratch.append(pltpu.VMEM_SHARED((table_rows, HALF), jnp.float32))

    @functools.partial(
        pl.kernel,
        out_type=jax.ShapeDtypeStruct((NC * NACC, HALF), jnp.float32),
        mesh=mesh,
        scratch_types=scratch,
        compiler_params=pltpu.CompilerParams(needs_layout_passes=False),
    )
    def sc_agg(table_hbm, pidx_hbm, zeros_hbm, out_hbm,
               pidx_v, comp_v, gidx_bufs, dst_bufs, rows_bufs,
               acc_sh, gsems, *maybe_tab):
        c = lax.axis_index("c")
        s = lax.axis_index("s")
        table = maybe_tab[0] if resident else table_hbm
        pltpu.sync_copy(pidx_hbm.at[s], pidx_v)
        pltpu.sync_copy(zeros_hbm, acc_sh.at[pl.ds(s * astripe, astripe)])
        if resident:
            pltpu.sync_copy(
                table_hbm.at[pl.ds(s * tstripe, tstripe)],
                maybe_tab[0].at[pl.ds(s * tstripe, tstripe)])
        plsc.subcore_barrier()

        trash = eps + NBUF * CHUNK + 15

        def scan_body(i, ptr):
            p16 = pidx_v[pl.ds(i * 16, 16)]
            keep = (p16 & PMASK) < N2
            prefix = plsc.cumsum(keep.astype(jnp.int32))
            pos = jnp.where(keep, ptr + prefix - 1, trash)
            plsc.store_scatter(comp_v, [pos], p16)
            return ptr + prefix[15]

        cnt = lax.fori_loop(0, eps // 16, scan_body, jnp.int32(0))
        for k in range(NBUF * CHUNK // 16):
            comp_v[pl.ds(cnt + k * 16, 16)] = jnp.full((16,), SCRAP, jnp.int32)
        n_ch = (cnt + CHUNK - 1) // CHUNK
        n_cp = jnp.maximum((n_ch + NBUF - 1) // NBUF * NBUF, NBUF)

        def prep_idx(j, b):
            for k in range(CHUNK // 16):
                p16 = comp_v[pl.ds(j * CHUNK + k * 16, 16)]
                gidx_bufs[b][pl.ds(k * 16, 16)] = 2 * (p16 >> PBITS) + c
                dst_bufs[b][pl.ds(k * 16, 16)] = p16 & PMASK

        def start_gather(b, sem):
            pltpu.async_copy(table.at[gidx_bufs[b]], rows_bufs[b], sem)

        def stage(j, b, sem):
            pltpu.make_async_copy(
                table.at[gidx_bufs[b]], rows_bufs[b], sem).wait()
            pltpu.sync_copy(rows_bufs[b], acc_sh.at[dst_bufs[b]], add=True)

            @pl.when(j + NBUF < n_cp)
            def _():
                prep_idx(j + NBUF, b)
                start_gather(b, sem)

        for b in range(NBUF):
            prep_idx(jnp.int32(b), b)
            start_gather(b, gsems[b])

        def ring_body(p, carry):
            for b in range(NBUF):
                stage(NBUF * p + b, b, gsems[b])
            return carry

        lax.fori_loop(0, n_cp // NBUF, ring_body, 0)
        plsc.subcore_barrier()
        pltpu.sync_copy(acc_sh.at[pl.ds(s * astripe, astripe)],
                        out_hbm.at[pl.ds(c * NACC + s * astripe, astripe)])

    return sc_agg


def _prep_edges(edge_index, eps):
    e = edge_index.astype(jnp.int32)
    pad = NS * eps - e.shape[1]
    p = (e[0] << PBITS) | e[1]
    p = jnp.concatenate([p, jnp.full((pad,), PMASK, jnp.int32)])
    return p.reshape(NS, eps)


def _mlp_body(x_ref, a0_ref, a1_ref, w1_ref, b1_ref, w2_ref, b2_ref, o_ref):
    h = x_ref[...] + jnp.concatenate([a0_ref[...], a1_ref[...]], axis=1)
    a = jnp.maximum(
        jnp.dot(h, w1_ref[...], preferred_element_type=jnp.float32)
        + b1_ref[...], 0.0)
    o_ref[...] = jnp.maximum(
        jnp.dot(a, w2_ref[...], preferred_element_type=jnp.float32)
        + b2_ref[...], 0.0)


def _mlp(x, aggbuf, n_rows, blk, W1, b1, W2, b2):
    grid = (NACC // blk,)
    nblk_off = NACC // blk
    return pl.pallas_call(
        _mlp_body,
        grid=grid,
        in_specs=[
            pl.BlockSpec((blk, D), lambda i: (i, 0)),
            pl.BlockSpec((blk, HALF), lambda i: (i, 0)),
            pl.BlockSpec((blk, HALF), lambda i, o=nblk_off: (o + i, 0)),
            pl.BlockSpec((D, D), lambda i: (0, 0)),
            pl.BlockSpec((D,), lambda i: (0,)),
            pl.BlockSpec((D, D), lambda i: (0, 0)),
            pl.BlockSpec((D,), lambda i: (0,)),
        ],
        out_specs=pl.BlockSpec((blk, D), lambda i: (i, 0)),
        out_shape=jax.ShapeDtypeStruct((n_rows, D), jnp.float32),
    )(x, aggbuf, aggbuf, W1, b1, W2, b2)


_EPS1 = 10240
_EPS2 = 2048
_H1ROWS = 2048

_sc_agg1 = _make_sc_agg(_EPS1, 2 * N1, resident=False, CHUNK=64, NBUF=8)
_sc_agg2 = _make_sc_agg(_EPS2, 2 * _H1ROWS, resident=True, CHUNK=128, NBUF=4)


def kernel(x, edge_index1, edge_index2, W1a, b1a, W2a, b2a,
           W1b, b1b, W2b, b2b):
    pidx1 = _prep_edges(edge_index1, _EPS1)
    pidx2 = _prep_edges(edge_index2, _EPS2)
    zeros = jnp.zeros((NACC // NS, HALF), jnp.float32)

    x2 = x.reshape(2 * N0, HALF)
    agg1 = _sc_agg1(x2, pidx1, zeros)
    h1 = _mlp(x, agg1, _H1ROWS, 512, W1a, b1a, W2a, b2a)

    h1_2 = h1.reshape(2 * _H1ROWS, HALF)
    agg2 = _sc_agg2(h1_2, pidx2, zeros)
    h2 = _mlp(h1, agg2, N2, 512, W1b, b1b, W2b, b2b)
    return h2


# speedup vs baseline: 1.0248x; 1.0183x over previous
import functools

import jax
import jax.numpy as jnp
from jax import lax
from jax.experimental import pallas as pl
from jax.experimental.pallas import tpu as pltpu
from jax.experimental.pallas import tpu_sc as plsc

N0, N1, N2 = 50000, 10000, 2000
D = 256
HALF = 128
E1, E2 = 160000, 32000

NC, NS = 2, 16
NACC = 2048
PBITS = 14
PMASK = (1 << PBITS) - 1
SCRAP = NACC - 1


def _make_sc_agg(eps, table_rows, resident, CHUNK, NBUF):
    astripe = NACC // NS
    tstripe = table_rows // NS
    mesh = plsc.VectorSubcoreMesh(
        core_axis_name="c", subcore_axis_name="s", num_cores=NC,
        num_subcores=NS)

    scratch = [
        pltpu.VMEM((eps,), jnp.int32),
        pltpu.VMEM((eps + NBUF * CHUNK + 16,), jnp.int32),
        [pltpu.VMEM((CHUNK,), jnp.int32) for _ in range(NBUF)],
        [pltpu.VMEM((CHUNK,), jnp.int32) for _ in range(NBUF)],
        [pltpu.VMEM((CHUNK, HALF), jnp.float32) for _ in range(NBUF)],
        pltpu.VMEM_SHARED((NACC, HALF), jnp.float32),
        [pltpu.SemaphoreType.DMA for _ in range(NBUF)],
    ]
    if resident:
        scratch.append(pltpu.VMEM_SHARED((table_rows, HALF), jnp.float32))

    @functools.partial(
        pl.kernel,
        out_type=jax.ShapeDtypeStruct((NC * NACC, HALF), jnp.float32),
        mesh=mesh,
        scratch_types=scratch,
        compiler_params=pltpu.CompilerParams(needs_layout_passes=False),
    )
    def sc_agg(table_hbm, pidx_hbm, zeros_hbm, out_hbm,
               pidx_v, comp_v, gidx_bufs, dst_bufs, rows_bufs,
               acc_sh, gsems, *maybe_tab):
        c = lax.axis_index("c")
        s = lax.axis_index("s")
        table = maybe_tab[0] if resident else table_hbm
        pltpu.sync_copy(pidx_hbm.at[s], pidx_v)
        pltpu.sync_copy(zeros_hbm, acc_sh.at[pl.ds(s * astripe, astripe)])
        if resident:
            pltpu.sync_copy(
                table_hbm.at[pl.ds(s * tstripe, tstripe)],
                maybe_tab[0].at[pl.ds(s * tstripe, tstripe)])
        plsc.subcore_barrier()

        trash = eps + NBUF * CHUNK + 15

        def scan_body(i, ptr):
            ps, keeps, prefixes = [], [], []
            for u in range(4):
                p16 = pidx_v[pl.ds(i * 64 + u * 16, 16)]
                keep = (p16 & PMASK) < N2
                ps.append(p16)
                keeps.append(keep)
                prefixes.append(plsc.cumsum(keep.astype(jnp.int32)))
            base = ptr
            for u in range(4):
                pos = jnp.where(keeps[u], base + prefixes[u] - 1, trash)
                plsc.store_scatter(comp_v, [pos], ps[u])
                base = base + prefixes[u][15]
            return base

        cnt = lax.fori_loop(0, eps // 64, scan_body, jnp.int32(0))
        for k in range(NBUF * CHUNK // 16):
            comp_v[pl.ds(cnt + k * 16, 16)] = jnp.full((16,), SCRAP, jnp.int32)
        n_ch = (cnt + CHUNK - 1) // CHUNK
        n_cp = jnp.maximum((n_ch + NBUF - 1) // NBUF * NBUF, NBUF)

        def prep_idx(j, b):
            for k in range(CHUNK // 16):
                p16 = comp_v[pl.ds(j * CHUNK + k * 16, 16)]
                gidx_bufs[b][pl.ds(k * 16, 16)] = 2 * (p16 >> PBITS) + c
                dst_bufs[b][pl.ds(k * 16, 16)] = p16 & PMASK

        def start_gather(b, sem):
            pltpu.async_copy(table.at[gidx_bufs[b]], rows_bufs[b], sem)

        def stage(j, b, sem):
            pltpu.make_async_copy(
                table.at[gidx_bufs[b]], rows_bufs[b], sem).wait()
            pltpu.sync_copy(rows_bufs[b], acc_sh.at[dst_bufs[b]], add=True)

            @pl.when(j + NBUF < n_cp)
            def _():
                prep_idx(j + NBUF, b)
                start_gather(b, sem)

        for b in range(NBUF):
            prep_idx(jnp.int32(b), b)
            start_gather(b, gsems[b])

        def ring_body(p, carry):
            for b in range(NBUF):
                stage(NBUF * p + b, b, gsems[b])
            return carry

        lax.fori_loop(0, n_cp // NBUF, ring_body, 0)
        plsc.subcore_barrier()
        pltpu.sync_copy(acc_sh.at[pl.ds(s * astripe, astripe)],
                        out_hbm.at[pl.ds(c * NACC + s * astripe, astripe)])

    return sc_agg


def _prep_edges(edge_index, eps):
    e = edge_index.astype(jnp.int32)
    pad = NS * eps - e.shape[1]
    p = (e[0] << PBITS) | e[1]
    p = jnp.concatenate([p, jnp.full((pad,), PMASK, jnp.int32)])
    return p.reshape(NS, eps)


def _mlp_body(x_ref, a0_ref, a1_ref, w1_ref, b1_ref, w2_ref, b2_ref, o_ref):
    h = x_ref[...] + jnp.concatenate([a0_ref[...], a1_ref[...]], axis=1)
    a = jnp.maximum(
        jnp.dot(h, w1_ref[...], preferred_element_type=jnp.float32)
        + b1_ref[...], 0.0)
    o_ref[...] = jnp.maximum(
        jnp.dot(a, w2_ref[...], preferred_element_type=jnp.float32)
        + b2_ref[...], 0.0)


def _mlp(x, aggbuf, n_rows, blk, W1, b1, W2, b2):
    grid = (NACC // blk,)
    nblk_off = NACC // blk
    return pl.pallas_call(
        _mlp_body,
        grid=grid,
        in_specs=[
            pl.BlockSpec((blk, D), lambda i: (i, 0)),
            pl.BlockSpec((blk, HALF), lambda i: (i, 0)),
            pl.BlockSpec((blk, HALF), lambda i, o=nblk_off: (o + i, 0)),
            pl.BlockSpec((D, D), lambda i: (0, 0)),
            pl.BlockSpec((D,), lambda i: (0,)),
            pl.BlockSpec((D, D), lambda i: (0, 0)),
            pl.BlockSpec((D,), lambda i: (0,)),
        ],
        out_specs=pl.BlockSpec((blk, D), lambda i: (i, 0)),
        out_shape=jax.ShapeDtypeStruct((n_rows, D), jnp.float32),
    )(x, aggbuf, aggbuf, W1, b1, W2, b2)


_EPS1 = 10240
_EPS2 = 2048
_H1ROWS = 2048

_sc_agg1 = _make_sc_agg(_EPS1, 2 * N1, resident=False, CHUNK=64, NBUF=8)
_sc_agg2 = _make_sc_agg(_EPS2, 2 * _H1ROWS, resident=True, CHUNK=128, NBUF=4)


def kernel(x, edge_index1, edge_index2, W1a, b1a, W2a, b2a,
           W1b, b1b, W2b, b2b):
    pidx1 = _prep_edges(edge_index1, _EPS1)
    pidx2 = _prep_edges(edge_index2, _EPS2)
    zeros = jnp.zeros((NACC // NS, HALF), jnp.float32)

    x2 = x.reshape(2 * N0, HALF)
    agg1 = _sc_agg1(x2, pidx1, zeros)
    h1 = _mlp(x, agg1, _H1ROWS, 512, W1a, b1a, W2a, b2a)

    h1_2 = h1.reshape(2 * _H1ROWS, HALF)
    agg2 = _sc_agg2(h1_2, pidx2, zeros)
    h2 = _mlp(h1, agg2, N2, 512, W1b, b1b, W2b, b2b)
    return h2
